# baseline (speedup 1.0000x reference)
import functools

import jax
import jax.numpy as jnp
from jax import lax
from jax.experimental import pallas as pl
from jax.experimental.pallas import tpu as pltpu
from jax.experimental.pallas import tpu_sc as plsc

_B, _C, _D = 128, 1024, 256
_NC, _NS = 2, 16
_NW = _NC * _NS
_BPW = _B // _NW
_CHUNK = 64
_CPB = _C // _CHUNK
_NBUF = 4
_T = _BPW * _CPB
_NGRP = _T // _NBUF


def _worker_body(x_hbm, inv_hbm, out_hbm, idx_v, inv_v, rows_v, gsems, wsems):
    wid = lax.axis_index("s") * _NC + lax.axis_index("c")
    pltpu.sync_copy(inv_hbm, idx_v)
    b0 = wid * _BPW

    def repack_body(j, carry):
        src = pl.multiple_of(j * 16, 16)
        inv_v[j // (_CHUNK // 16), pl.ds(pl.multiple_of((j % (_CHUNK // 16)) * 16, 16), 16)] = idx_v[pl.ds(src, 16)]
        return carry

    lax.fori_loop(0, _C // 16, repack_body, 0)

    def read(t, slot):
        b = b0 + t // _CPB
        koff = pl.multiple_of((t % _CPB) * _CHUNK, _CHUNK)
        pltpu.async_copy(
            x_hbm.at[b].at[pl.ds(koff, _CHUNK)],
            rows_v.at[slot],
            gsems[slot],
        )

    def wait_read(slot):
        pltpu.make_async_copy(
            x_hbm.at[0].at[pl.ds(0, _CHUNK)],
            rows_v.at[slot],
            gsems[slot],
        ).wait()

    def scatter(t, slot):
        b = b0 + t // _CPB
        k = t % _CPB
        pltpu.async_copy(
            rows_v.at[slot],
            out_hbm.at[b].at[inv_v.at[k]],
            wsems[slot],
        )

    def wait_scatter(slot):
        pltpu.make_async_copy(
            rows_v.at[slot],
            out_hbm.at[0].at[pl.ds(0, _CHUNK)],
            wsems[slot],
        ).wait()

    def group_body(g, carry):
        t0 = g * _NBUF
        for s in range(_NBUF):

            @pl.when(g > 0)
            def _():
                wait_scatter(s)

            read(t0 + s, s)
            prev = (s - 2) % _NBUF
            if s <= 1:

                @pl.when(g > 0)
                def _():
                    wait_read(prev)
                    scatter(t0 + s - 2, prev)

            else:
                wait_read(prev)
                scatter(t0 + s - 2, prev)
        return carry

    lax.fori_loop(0, _NGRP, group_body, 0)
    for s in (_NBUF - 2, _NBUF - 1):
        wait_read(s)
        scatter(_T - _NBUF + s, s)
    for s in range(_NBUF):
        wait_scatter(s)


@jax.jit
def _sc_subsample(x, idx):
    mesh = plsc.VectorSubcoreMesh(core_axis_name="c", subcore_axis_name="s")
    f = pl.kernel(
        _worker_body,
        mesh=mesh,
        out_type=jax.ShapeDtypeStruct((_B, _C, _D), jnp.float32),
        scratch_types=[
            pltpu.VMEM((_C,), jnp.int32),
            pltpu.VMEM((_CPB, _CHUNK), jnp.int32),
            pltpu.VMEM((_NBUF, _CHUNK, _D), jnp.float32),
            [pltpu.SemaphoreType.DMA] * _NBUF,
            [pltpu.SemaphoreType.DMA] * _NBUF,
        ],
    )
    return f(x, idx)


def kernel(x, forward_shuffle_idx):
    inv = (
        jnp.zeros((_C,), jnp.int32)
        .at[forward_shuffle_idx]
        .set(jnp.arange(_C, dtype=jnp.int32))
    )
    return _sc_subsample(x, inv)
